# speedup vs baseline: 1.9694x; 1.9694x over previous
import functools

import jax
import jax.numpy as jnp
from jax import lax
from jax.experimental import pallas as pl
from jax.experimental.pallas import tpu as pltpu

N = 10000
D = 128
HID = 128
OUT = 16
L = 2
R = 5
NPAD = 10240

REL = ((0, 0), (0, 1), (2, 2), (0, 2), (0, 3))


def _mm_body(x_ref, w_ref, o_ref):
    o_ref[...] = jnp.dot(x_ref[...], w_ref[...], preferred_element_type=jnp.float32)


def _mm(x, w, bn=1024):
    n, k = x.shape
    m = w.shape[1]
    return pl.pallas_call(
        _mm_body,
        grid=(n // bn,),
        in_specs=[
            pl.BlockSpec((bn, k), lambda i: (i, 0)),
            pl.BlockSpec((k, m), lambda i: (0, 0)),
        ],
        out_specs=pl.BlockSpec((bn, m), lambda i: (i, 0)),
        out_shape=jax.ShapeDtypeStruct((n, m), jnp.float32),
    )(x, w)


def _pad_rows(x):
    return jnp.concatenate([x, jnp.zeros((NPAD - x.shape[0], x.shape[1]), x.dtype)], 0)


def _edge_phase(ls, ld, src, dst, x_src, m_shift):
    e = ls[src] + ld[dst]
    e = jnp.maximum(e, 0.2 * e)
    ex = jnp.exp(e - m_shift)
    denom = jax.ops.segment_sum(ex, dst, num_segments=N)
    agg = jax.ops.segment_sum(x_src[src] * ex[:, None], dst, num_segments=N)
    return agg, denom


def kernel(x_activity, x_resource_static, x_resource_dynamic, x_attribute,
           ei_follows, ei_has_rs, ei_rdelta, ei_has_rd, ei_has_attr,
           Wsrc, Wdst, Asrc, Adst, Bias, Wln, bln, Wfc, bfc):
    eis = (ei_follows, ei_has_rs, ei_rdelta, ei_has_rd, ei_has_attr)
    xs = [x_activity, x_resource_static, x_resource_dynamic, x_attribute]
    wsv = jnp.einsum('lrdh,lrh->lrd', Wsrc, Asrc)
    wdv = jnp.einsum('lrdh,lrh->lrd', Wdst, Adst)

    for l in range(L):
        cols = {t: [] for t in range(4)}
        for r in range(R):
            st, dt = REL[r]
            cols[st].append(('s', r, wsv[l, r]))
            cols[dt].append(('d', r, wdv[l, r]))
        ls = [None] * R
        ld = [None] * R
        for t in range(4):
            if not cols[t]:
                continue
            V = jnp.stack([c[2] for c in cols[t]], axis=1)
            Vp = jnp.concatenate([V, jnp.zeros((D, 128 - V.shape[1]), V.dtype)], 1)
            al = _mm(_pad_rows(xs[t]), Vp)[:N]
            for j, (kind, r, _) in enumerate(cols[t]):
                if kind == 's':
                    ls[r] = al[:, j]
                else:
                    ld[r] = al[:, j]

        outs = []
        for r in range(R):
            st, dt = REL[r]
            m_shift = jnp.max(ls[r]) + jnp.max(ld[r])
            agg, denom = _edge_phase(ls[r], ld[r], eis[r][0], eis[r][1], xs[st], m_shift)
            pooled = agg / (denom[:, None] + 1e-30)
            outs.append(_mm(_pad_rows(pooled), Wsrc[l, r])[:N] + Bias[l, r])
        xs = [
            jax.nn.relu(outs[0]),
            jax.nn.relu(outs[1]),
            jax.nn.relu((outs[2] + outs[3]) * 0.5),
            jax.nn.relu(outs[4]),
        ]

    feats = []
    for t in range(4):
        h = _mm(_pad_rows(xs[t]), Wln)[:N] + bln
        feats.append(jnp.mean(jax.nn.relu(h), axis=0))
    cat = jnp.concatenate(feats)
    logits = cat @ Wfc + bfc
    return jax.nn.softmax(logits, axis=-1)


# speedup vs baseline: 15.0902x; 7.6622x over previous
import functools

import jax
import jax.numpy as jnp
from jax import lax
from jax.experimental import pallas as pl
from jax.experimental.pallas import tpu as pltpu
from jax.experimental.pallas import tpu_sc as plsc

N = 10000
D = 128
HID = 128
OUT = 16
L = 2
R = 5
NPAD = 10240

NC = 2
NS = 16
HD = D // 2
ECH = 128
QCH = 40
NQ = 4
CHUNKS = QCH * NQ
EPT = CHUNKS * ECH
NROWS = NPAD // 16
RSTRIDE = NPAD // NS
DSTRIDE = NROWS // NS

_sc_mesh = plsc.VectorSubcoreMesh(
    core_axis_name="c", subcore_axis_name="s", num_cores=NC, num_subcores=NS)


@functools.partial(
    pl.kernel,
    out_type=(jax.ShapeDtypeStruct((NC, NPAD, HD), jnp.float32),
              jax.ShapeDtypeStruct((NC, NROWS, 16), jnp.float32)),
    mesh=_sc_mesh,
    compiler_params=pltpu.CompilerParams(
        needs_layout_passes=False, use_tc_tiling_on_sc=False),
    scratch_types=(
        pltpu.VMEM((NPAD,), jnp.float32),
        pltpu.VMEM((NPAD,), jnp.float32),
        pltpu.VMEM((QCH, ECH), jnp.int32),
        pltpu.VMEM((QCH, ECH), jnp.int32),
        pltpu.VMEM((ECH,), jnp.float32),
        pltpu.VMEM((ECH, HD), jnp.float32),
        pltpu.VMEM((NROWS, 16), jnp.float32),
        pltpu.VMEM((16,), jnp.float32),
        pltpu.VMEM((5, ECH), jnp.int32),
        pltpu.VMEM_SHARED((NPAD, HD), jnp.float32),
        pltpu.VMEM_SHARED((NROWS, 16), jnp.float32),
        pltpu.SemaphoreType.DMA,
    ),
)
def _edge_sc(x_hbm, ls_hbm, ld_hbm, m_hbm, src_hbm, dst_hbm,
             agg_out, den_out,
             ls_v, ld_v, src_v, dst_v, ex_v, rows_v, den_v, m_v, iota_v,
             agg_s, den_s, sem):
    c = lax.axis_index("c")
    s = lax.axis_index("s")

    pltpu.sync_copy(ls_hbm, ls_v)
    pltpu.sync_copy(ld_hbm, ld_v)
    pltpu.sync_copy(m_hbm, m_v)

    zero16f = jnp.zeros((16,), jnp.float32)
    iota16 = lax.iota(jnp.int32, 16)

    def _zrow(i, _):
        for f in range(HD // 16):
            rows_v[i, pl.ds(f * 16, 16)] = zero16f
        return ()
    lax.fori_loop(0, ECH, _zrow, ())

    def _zden(i, _):
        den_v[i, :] = zero16f
        return ()
    lax.fori_loop(0, NROWS, _zden, ())

    def _ziota(k, _):
        iota_v[k // 8, pl.ds((k % 8) * 16, 16)] = iota16 + k * 16
        return ()
    lax.fori_loop(0, 40, _ziota, ())

    for j in range(RSTRIDE // ECH):
        pltpu.sync_copy(rows_v, agg_s.at[pl.ds(s * RSTRIDE + j * ECH, ECH)])
    pltpu.sync_copy(den_v.at[pl.ds(s * DSTRIDE, DSTRIDE)],
                    den_s.at[pl.ds(s * DSTRIDE, DSTRIDE)])
    plsc.subcore_barrier()

    m = m_v[...]

    def chunk_body(ci, _):
        for v in range(8):
            s16 = src_v[ci, pl.ds(v * 16, 16)]
            d16 = dst_v[ci, pl.ds(v * 16, 16)]
            als = plsc.load_gather(ls_v, [s16])
            ald = plsc.load_gather(ld_v, [d16])
            e = als + ald
            e = jnp.maximum(e, e * 0.2)
            ex = jnp.exp(e - m)
            ex_v[pl.ds(v * 16, 16)] = ex
            plsc.addupdate_scatter(
                den_v, [lax.shift_right_logical(d16, 4),
                        lax.bitwise_and(d16, 15)], ex)
        pltpu.async_copy(x_hbm.at[c].at[src_v.at[ci]], rows_v, sem).wait()

        def row_body(i, _):
            spl = plsc.load_gather(ex_v, [jnp.zeros((16,), jnp.int32) + i])
            for f in range(HD // 16):
                rows_v[i, pl.ds(f * 16, 16)] = rows_v[i, pl.ds(f * 16, 16)] * spl
            return ()
        lax.fori_loop(0, ECH, row_body, ())

        pltpu.sync_copy(rows_v, agg_s.at[dst_v.at[ci]], add=True)
        return ()

    for q in range(NQ):
        pltpu.sync_copy(src_hbm.at[s, pl.ds(q * QCH, QCH)], src_v)
        pltpu.sync_copy(dst_hbm.at[s, pl.ds(q * QCH, QCH)], dst_v)
        lax.fori_loop(0, QCH, chunk_body, ())

    plsc.subcore_barrier()
    for j in range(5):
        pltpu.sync_copy(den_v.at[pl.ds(j * ECH, ECH)],
                        den_s.at[iota_v.at[j]], add=True)
    plsc.subcore_barrier()

    pltpu.sync_copy(agg_s.at[pl.ds(s * RSTRIDE, RSTRIDE)],
                    agg_out.at[c, pl.ds(s * RSTRIDE, RSTRIDE)])
    pltpu.sync_copy(den_s.at[pl.ds(s * DSTRIDE, DSTRIDE)],
                    den_out.at[c, pl.ds(s * DSTRIDE, DSTRIDE)])

REL = ((0, 0), (0, 1), (2, 2), (0, 2), (0, 3))


def _mm_body(x_ref, w_ref, o_ref):
    o_ref[...] = jnp.dot(x_ref[...], w_ref[...], preferred_element_type=jnp.float32)


def _mm(x, w, bn=1024):
    n, k = x.shape
    m = w.shape[1]
    return pl.pallas_call(
        _mm_body,
        grid=(n // bn,),
        in_specs=[
            pl.BlockSpec((bn, k), lambda i: (i, 0)),
            pl.BlockSpec((k, m), lambda i: (0, 0)),
        ],
        out_specs=pl.BlockSpec((bn, m), lambda i: (i, 0)),
        out_shape=jax.ShapeDtypeStruct((n, m), jnp.float32),
    )(x, w)


def _pad_rows(x):
    return jnp.concatenate([x, jnp.zeros((NPAD - x.shape[0], x.shape[1]), x.dtype)], 0)


def _pad_edges(ei):
    pad = jnp.full((2, NS * EPT - ei.shape[1]), NPAD - 1, jnp.int32)
    e = jnp.concatenate([ei, pad], axis=1)
    return e[0].reshape(NS, CHUNKS, ECH), e[1].reshape(NS, CHUNKS, ECH)


def _pad_vec(v):
    return jnp.concatenate([v, jnp.full((NPAD - v.shape[0],), -1e9, v.dtype)])


def _edge_phase(ls, ld, src3, dst3, x2):
    m_shift = jnp.max(ls) + jnp.max(ld)
    m_splat = jnp.full((16,), m_shift, jnp.float32)
    agg2, den2 = _edge_sc(x2, _pad_vec(ls), _pad_vec(ld), m_splat,
                          src3, dst3)
    agg = jnp.concatenate([agg2[0], agg2[1]], axis=1)
    den = den2[0].reshape(NPAD)
    return agg / (den[:, None] + 1e-30)


def kernel(x_activity, x_resource_static, x_resource_dynamic, x_attribute,
           ei_follows, ei_has_rs, ei_rdelta, ei_has_rd, ei_has_attr,
           Wsrc, Wdst, Asrc, Adst, Bias, Wln, bln, Wfc, bfc):
    eis = [_pad_edges(ei) for ei in
           (ei_follows, ei_has_rs, ei_rdelta, ei_has_rd, ei_has_attr)]
    xs = [x_activity, x_resource_static, x_resource_dynamic, x_attribute]
    wsv = jnp.einsum('lrdh,lrh->lrd', Wsrc, Asrc)
    wdv = jnp.einsum('lrdh,lrh->lrd', Wdst, Adst)

    for l in range(L):
        cols = {t: [] for t in range(4)}
        for r in range(R):
            st, dt = REL[r]
            cols[st].append(('s', r, wsv[l, r]))
            cols[dt].append(('d', r, wdv[l, r]))
        ls = [None] * R
        ld = [None] * R
        for t in range(4):
            if not cols[t]:
                continue
            V = jnp.stack([c[2] for c in cols[t]], axis=1)
            Vp = jnp.concatenate([V, jnp.zeros((D, 128 - V.shape[1]), V.dtype)], 1)
            al = _mm(_pad_rows(xs[t]), Vp)[:N]
            for j, (kind, r, _) in enumerate(cols[t]):
                if kind == 's':
                    ls[r] = al[:, j]
                else:
                    ld[r] = al[:, j]

        xpads = {}
        for r in range(R):
            st = REL[r][0]
            if st not in xpads:
                xp = _pad_rows(xs[st])
                xpads[st] = jnp.stack([xp[:, :HD], xp[:, HD:]])
        outs = []
        for r in range(R):
            st, dt = REL[r]
            pooled = _edge_phase(ls[r], ld[r], eis[r][0], eis[r][1], xpads[st])
            outs.append(_mm(pooled, Wsrc[l, r])[:N] + Bias[l, r])
        xs = [
            jax.nn.relu(outs[0]),
            jax.nn.relu(outs[1]),
            jax.nn.relu((outs[2] + outs[3]) * 0.5),
            jax.nn.relu(outs[4]),
        ]

    feats = []
    for t in range(4):
        h = _mm(_pad_rows(xs[t]), Wln)[:N] + bln
        feats.append(jnp.mean(jax.nn.relu(h), axis=0))
    cat = jnp.concatenate(feats)
    logits = cat @ Wfc + bfc
    return jax.nn.softmax(logits, axis=-1)


# speedup vs baseline: 23.3402x; 1.5467x over previous
import functools

import jax
import jax.numpy as jnp
from jax import lax
from jax.experimental import pallas as pl
from jax.experimental.pallas import tpu as pltpu
from jax.experimental.pallas import tpu_sc as plsc

N = 10000
D = 128
HID = 128
OUT = 16
L = 2
R = 5
NPAD = 10240

NC = 2
NS = 16
HD = D // 2
ECH = 128
CHUNKS = 160
EPT = CHUNKS * ECH
NROWS = NPAD // 16
RSTRIDE = NPAD // NS
DSTRIDE = NROWS // NS

_sc_mesh = plsc.VectorSubcoreMesh(
    core_axis_name="c", subcore_axis_name="s", num_cores=NC, num_subcores=NS)


@functools.partial(
    pl.kernel,
    out_type=(jax.ShapeDtypeStruct((NC, NPAD, HD), jnp.float32),
              jax.ShapeDtypeStruct((NC, NROWS, 16), jnp.float32)),
    mesh=_sc_mesh,
    compiler_params=pltpu.CompilerParams(
        needs_layout_passes=False, use_tc_tiling_on_sc=False),
    scratch_types=(
        pltpu.VMEM((NPAD,), jnp.float32),
        pltpu.VMEM((NPAD,), jnp.float32),
        pltpu.VMEM((CHUNKS, ECH), jnp.int32),
        pltpu.VMEM((CHUNKS, ECH), jnp.int32),
        pltpu.VMEM((ECH,), jnp.float32),
        pltpu.VMEM((2, ECH, HD), jnp.float32),
        pltpu.VMEM((NROWS, 16), jnp.float32),
        pltpu.VMEM((16,), jnp.float32),
        pltpu.VMEM((5, ECH), jnp.int32),
        pltpu.VMEM_SHARED((NPAD, HD), jnp.float32),
        pltpu.VMEM_SHARED((NROWS, 16), jnp.float32),
        pltpu.SemaphoreType.DMA,
        pltpu.SemaphoreType.DMA,
        pltpu.SemaphoreType.DMA,
        pltpu.SemaphoreType.DMA,
    ),
)
def _edge_sc(x_hbm, ls_hbm, ld_hbm, m_hbm, src_hbm, dst_hbm,
             agg_out, den_out,
             ls_v, ld_v, src_v, dst_v, ex_v, rows_v, den_v, m_v, iota_v,
             agg_s, den_s, sem_g0, sem_g1, sem_s0, sem_s1):
    c = lax.axis_index("c")
    s = lax.axis_index("s")

    pltpu.sync_copy(ls_hbm, ls_v)
    pltpu.sync_copy(ld_hbm, ld_v)
    pltpu.sync_copy(m_hbm, m_v)
    pltpu.sync_copy(src_hbm.at[s], src_v)
    pltpu.sync_copy(dst_hbm.at[s], dst_v)

    zero16f = jnp.zeros((16,), jnp.float32)
    iota16 = lax.iota(jnp.int32, 16)

    def _zrow(i, _):
        for f in range(HD // 16):
            rows_v[0, i, pl.ds(f * 16, 16)] = zero16f
        return ()
    lax.fori_loop(0, ECH, _zrow, ())

    def _zden(i, _):
        den_v[i, :] = zero16f
        return ()
    lax.fori_loop(0, NROWS, _zden, ())

    def _ziota(k, _):
        iota_v[k // 8, pl.ds((k % 8) * 16, 16)] = iota16 + k * 16
        return ()
    lax.fori_loop(0, 40, _ziota, ())

    for j in range(RSTRIDE // ECH):
        pltpu.sync_copy(rows_v.at[0], agg_s.at[pl.ds(s * RSTRIDE + j * ECH, ECH)])
    pltpu.sync_copy(den_v.at[pl.ds(s * DSTRIDE, DSTRIDE)],
                    den_s.at[pl.ds(s * DSTRIDE, DSTRIDE)])
    plsc.subcore_barrier()

    m = m_v[...]

    pltpu.async_copy(x_hbm.at[c].at[src_v.at[0]], rows_v.at[0], sem_g0)

    @pl.loop(0, CHUNKS, step=2)
    def _pipe(ci):
        for b in range(2):
            cc = ci + b
            sg_cur = sem_g0 if b == 0 else sem_g1
            sg_oth = sem_g1 if b == 0 else sem_g0
            ss_cur = sem_s0 if b == 0 else sem_s1
            ss_oth = sem_s1 if b == 0 else sem_s0
            rb_cur = rows_v.at[b]
            rb_oth = rows_v.at[1 - b]

            @pl.when(cc >= 1)
            def _():
                pltpu.make_async_copy(rb_oth, agg_s.at[dst_v.at[cc - 1]],
                                      ss_oth).wait()

            @pl.when(cc + 1 < CHUNKS)
            def _():
                pltpu.async_copy(x_hbm.at[c].at[src_v.at[cc + 1]], rb_oth,
                                 sg_oth)

            for v in range(8):
                s16 = src_v[cc, pl.ds(v * 16, 16)]
                d16 = dst_v[cc, pl.ds(v * 16, 16)]
                als = plsc.load_gather(ls_v, [s16])
                ald = plsc.load_gather(ld_v, [d16])
                e = als + ald
                e = jnp.maximum(e, e * 0.2)
                ex = jnp.exp(e - m)
                ex_v[pl.ds(v * 16, 16)] = ex
                plsc.addupdate_scatter(
                    den_v, [lax.shift_right_logical(d16, 4),
                            lax.bitwise_and(d16, 15)], ex)

            pltpu.make_async_copy(x_hbm.at[c].at[src_v.at[cc]], rb_cur,
                                  sg_cur).wait()

            def row_body(i, _):
                spl = plsc.load_gather(ex_v, [jnp.zeros((16,), jnp.int32) + i])
                for f in range(HD // 16):
                    rows_v[b, i, pl.ds(f * 16, 16)] = (
                        rows_v[b, i, pl.ds(f * 16, 16)] * spl)
                return ()
            lax.fori_loop(0, ECH, row_body, (), unroll=4)

            pltpu.async_copy(rb_cur, agg_s.at[dst_v.at[cc]], ss_cur, add=True)

    pltpu.make_async_copy(rows_v.at[1], agg_s.at[dst_v.at[0]], sem_s1).wait()

    plsc.subcore_barrier()
    for j in range(5):
        pltpu.sync_copy(den_v.at[pl.ds(j * ECH, ECH)],
                        den_s.at[iota_v.at[j]], add=True)
    plsc.subcore_barrier()

    pltpu.sync_copy(agg_s.at[pl.ds(s * RSTRIDE, RSTRIDE)],
                    agg_out.at[c, pl.ds(s * RSTRIDE, RSTRIDE)])
    pltpu.sync_copy(den_s.at[pl.ds(s * DSTRIDE, DSTRIDE)],
                    den_out.at[c, pl.ds(s * DSTRIDE, DSTRIDE)])

REL = ((0, 0), (0, 1), (2, 2), (0, 2), (0, 3))


def _mm_body(x_ref, w_ref, o_ref):
    o_ref[...] = jnp.dot(x_ref[...], w_ref[...], preferred_element_type=jnp.float32)


def _mm(x, w, bn=1024):
    n, k = x.shape
    m = w.shape[1]
    return pl.pallas_call(
        _mm_body,
        grid=(n // bn,),
        in_specs=[
            pl.BlockSpec((bn, k), lambda i: (i, 0)),
            pl.BlockSpec((k, m), lambda i: (0, 0)),
        ],
        out_specs=pl.BlockSpec((bn, m), lambda i: (i, 0)),
        out_shape=jax.ShapeDtypeStruct((n, m), jnp.float32),
    )(x, w)


def _pad_rows(x):
    return jnp.concatenate([x, jnp.zeros((NPAD - x.shape[0], x.shape[1]), x.dtype)], 0)


def _pad_edges(ei):
    pad = jnp.full((2, NS * EPT - ei.shape[1]), NPAD - 1, jnp.int32)
    e = jnp.concatenate([ei, pad], axis=1)
    return e[0].reshape(NS, CHUNKS, ECH), e[1].reshape(NS, CHUNKS, ECH)


def _pad_vec(v):
    return jnp.concatenate([v, jnp.full((NPAD - v.shape[0],), -1e9, v.dtype)])


def _edge_phase(ls, ld, src3, dst3, x2):
    m_shift = jnp.max(ls) + jnp.max(ld)
    m_splat = jnp.full((16,), m_shift, jnp.float32)
    agg2, den2 = _edge_sc(x2, _pad_vec(ls), _pad_vec(ld), m_splat,
                          src3, dst3)
    agg = jnp.concatenate([agg2[0], agg2[1]], axis=1)
    den = den2[0].reshape(NPAD)
    return agg / (den[:, None] + 1e-30)


def kernel(x_activity, x_resource_static, x_resource_dynamic, x_attribute,
           ei_follows, ei_has_rs, ei_rdelta, ei_has_rd, ei_has_attr,
           Wsrc, Wdst, Asrc, Adst, Bias, Wln, bln, Wfc, bfc):
    eis = [_pad_edges(ei) for ei in
           (ei_follows, ei_has_rs, ei_rdelta, ei_has_rd, ei_has_attr)]
    xs = [x_activity, x_resource_static, x_resource_dynamic, x_attribute]
    wsv = jnp.einsum('lrdh,lrh->lrd', Wsrc, Asrc)
    wdv = jnp.einsum('lrdh,lrh->lrd', Wdst, Adst)

    for l in range(L):
        cols = {t: [] for t in range(4)}
        for r in range(R):
            st, dt = REL[r]
            cols[st].append(('s', r, wsv[l, r]))
            cols[dt].append(('d', r, wdv[l, r]))
        ls = [None] * R
        ld = [None] * R
        for t in range(4):
            if not cols[t]:
                continue
            V = jnp.stack([c[2] for c in cols[t]], axis=1)
            Vp = jnp.concatenate([V, jnp.zeros((D, 128 - V.shape[1]), V.dtype)], 1)
            al = _mm(_pad_rows(xs[t]), Vp)[:N]
            for j, (kind, r, _) in enumerate(cols[t]):
                if kind == 's':
                    ls[r] = al[:, j]
                else:
                    ld[r] = al[:, j]

        xpads = {}
        for r in range(R):
            st = REL[r][0]
            if st not in xpads:
                xp = _pad_rows(xs[st])
                xpads[st] = jnp.stack([xp[:, :HD], xp[:, HD:]])
        outs = []
        for r in range(R):
            st, dt = REL[r]
            pooled = _edge_phase(ls[r], ld[r], eis[r][0], eis[r][1], xpads[st])
            outs.append(_mm(pooled, Wsrc[l, r])[:N] + Bias[l, r])
        xs = [
            jax.nn.relu(outs[0]),
            jax.nn.relu(outs[1]),
            jax.nn.relu((outs[2] + outs[3]) * 0.5),
            jax.nn.relu(outs[4]),
        ]

    feats = []
    for t in range(4):
        h = _mm(_pad_rows(xs[t]), Wln)[:N] + bln
        feats.append(jnp.mean(jax.nn.relu(h), axis=0))
    cat = jnp.concatenate(feats)
    logits = cat @ Wfc + bfc
    return jax.nn.softmax(logits, axis=-1)


# speedup vs baseline: 35.4697x; 1.5197x over previous
import functools

import jax
import jax.numpy as jnp
from jax import lax
from jax.experimental import pallas as pl
from jax.experimental.pallas import tpu as pltpu
from jax.experimental.pallas import tpu_sc as plsc

N = 10000
D = 128
HID = 128
OUT = 16
L = 2
R = 5
NPAD = 10240

NC = 2
NS = 16
HD = D // 2
ECH = 128
CHUNKS = 160
EPT = CHUNKS * ECH
NROWS = NPAD // 16
RSTRIDE = NPAD // NS
DSTRIDE = NROWS // NS

_sc_mesh = plsc.VectorSubcoreMesh(
    core_axis_name="c", subcore_axis_name="s", num_cores=NC, num_subcores=NS)


@functools.partial(
    pl.kernel,
    out_type=(jax.ShapeDtypeStruct((NC, NPAD, HD), jnp.bfloat16),
              jax.ShapeDtypeStruct((NC, NROWS, 16), jnp.float32)),
    mesh=_sc_mesh,
    compiler_params=pltpu.CompilerParams(
        needs_layout_passes=False, use_tc_tiling_on_sc=False),
    scratch_types=(
        pltpu.VMEM((NPAD,), jnp.float32),
        pltpu.VMEM((NPAD,), jnp.float32),
        pltpu.VMEM((CHUNKS, ECH), jnp.int32),
        pltpu.VMEM((CHUNKS, ECH), jnp.int32),
        pltpu.VMEM((ECH,), jnp.float32),
        pltpu.VMEM((2, ECH, HD), jnp.bfloat16),
        pltpu.VMEM((NROWS, 16), jnp.float32),
        pltpu.VMEM((16,), jnp.float32),
        pltpu.VMEM((5, ECH), jnp.int32),
        pltpu.VMEM_SHARED((NPAD, HD), jnp.bfloat16),
        pltpu.VMEM_SHARED((NROWS, 16), jnp.float32),
        pltpu.SemaphoreType.DMA,
        pltpu.SemaphoreType.DMA,
        pltpu.SemaphoreType.DMA,
        pltpu.SemaphoreType.DMA,
    ),
)
def _edge_sc(x_hbm, ls_hbm, ld_hbm, m_hbm, src_hbm, dst_hbm,
             agg_out, den_out,
             ls_v, ld_v, src_v, dst_v, ex_v, rows_v, den_v, m_v, iota_v,
             agg_s, den_s, sem_g0, sem_g1, sem_s0, sem_s1):
    c = lax.axis_index("c")
    s = lax.axis_index("s")

    pltpu.sync_copy(ls_hbm, ls_v)
    pltpu.sync_copy(ld_hbm, ld_v)
    pltpu.sync_copy(m_hbm, m_v)
    pltpu.sync_copy(src_hbm.at[s], src_v)
    pltpu.sync_copy(dst_hbm.at[s], dst_v)

    zero16f = jnp.zeros((16,), jnp.float32)
    iota16 = lax.iota(jnp.int32, 16)

    zero32bf = jnp.zeros((32,), jnp.bfloat16)

    def _zrow(i, _):
        for f in range(HD // 32):
            rows_v[0, i, pl.ds(f * 32, 32)] = zero32bf
        return ()
    lax.fori_loop(0, ECH, _zrow, ())

    def _zden(i, _):
        den_v[i, :] = zero16f
        return ()
    lax.fori_loop(0, NROWS, _zden, ())

    def _ziota(k, _):
        iota_v[k // 8, pl.ds((k % 8) * 16, 16)] = iota16 + k * 16
        return ()
    lax.fori_loop(0, 40, _ziota, ())

    for j in range(RSTRIDE // ECH):
        pltpu.sync_copy(rows_v.at[0], agg_s.at[pl.ds(s * RSTRIDE + j * ECH, ECH)])
    pltpu.sync_copy(den_v.at[pl.ds(s * DSTRIDE, DSTRIDE)],
                    den_s.at[pl.ds(s * DSTRIDE, DSTRIDE)])
    plsc.subcore_barrier()

    m = m_v[...]

    pltpu.async_copy(x_hbm.at[c].at[src_v.at[0]], rows_v.at[0], sem_g0)

    @pl.loop(0, CHUNKS, step=2)
    def _pipe(ci):
        for b in range(2):
            cc = ci + b
            sg_cur = sem_g0 if b == 0 else sem_g1
            sg_oth = sem_g1 if b == 0 else sem_g0
            ss_cur = sem_s0 if b == 0 else sem_s1
            ss_oth = sem_s1 if b == 0 else sem_s0
            rb_cur = rows_v.at[b]
            rb_oth = rows_v.at[1 - b]

            @pl.when(cc >= 1)
            def _():
                pltpu.make_async_copy(rb_oth, agg_s.at[dst_v.at[cc - 1]],
                                      ss_oth).wait()

            @pl.when(cc + 1 < CHUNKS)
            def _():
                pltpu.async_copy(x_hbm.at[c].at[src_v.at[cc + 1]], rb_oth,
                                 sg_oth)

            for v in range(8):
                s16 = src_v[cc, pl.ds(v * 16, 16)]
                d16 = dst_v[cc, pl.ds(v * 16, 16)]
                als = plsc.load_gather(ls_v, [s16])
                ald = plsc.load_gather(ld_v, [d16])
                e = als + ald
                e = jnp.maximum(e, e * 0.2)
                ex = jnp.exp(e - m)
                ex_v[pl.ds(v * 16, 16)] = ex
                plsc.addupdate_scatter(
                    den_v, [lax.shift_right_logical(d16, 4),
                            lax.bitwise_and(d16, 15)], ex)

            pltpu.make_async_copy(x_hbm.at[c].at[src_v.at[cc]], rb_cur,
                                  sg_cur).wait()

            def row_body(i, _):
                spl = plsc.load_gather(ex_v, [jnp.zeros((16,), jnp.int32) + i])
                for f in range(HD // 32):
                    v32 = rows_v[b, i, pl.ds(f * 32, 32)]
                    lo, hi = plsc.unpack(v32, format=plsc.PackFormat.INTERLEAVED)
                    rows_v[b, i, pl.ds(f * 32, 32)] = plsc.pack(
                        lo * spl, hi * spl,
                        format=plsc.PackFormat.INTERLEAVED)
                return ()
            lax.fori_loop(0, ECH, row_body, (), unroll=4)

            pltpu.async_copy(rb_cur, agg_s.at[dst_v.at[cc]], ss_cur, add=True)

    pltpu.make_async_copy(rows_v.at[1], agg_s.at[dst_v.at[0]], sem_s1).wait()

    plsc.subcore_barrier()
    for j in range(5):
        pltpu.sync_copy(den_v.at[pl.ds(j * ECH, ECH)],
                        den_s.at[iota_v.at[j]], add=True)
    plsc.subcore_barrier()

    pltpu.sync_copy(agg_s.at[pl.ds(s * RSTRIDE, RSTRIDE)],
                    agg_out.at[c, pl.ds(s * RSTRIDE, RSTRIDE)])
    pltpu.sync_copy(den_s.at[pl.ds(s * DSTRIDE, DSTRIDE)],
                    den_out.at[c, pl.ds(s * DSTRIDE, DSTRIDE)])

REL = ((0, 0), (0, 1), (2, 2), (0, 2), (0, 3))


def _mm_body(x_ref, w_ref, o_ref):
    o_ref[...] = jnp.dot(x_ref[...], w_ref[...], preferred_element_type=jnp.float32)


def _mm(x, w, bn=1024):
    n, k = x.shape
    m = w.shape[1]
    return pl.pallas_call(
        _mm_body,
        grid=(n // bn,),
        in_specs=[
            pl.BlockSpec((bn, k), lambda i: (i, 0)),
            pl.BlockSpec((k, m), lambda i: (0, 0)),
        ],
        out_specs=pl.BlockSpec((bn, m), lambda i: (i, 0)),
        out_shape=jax.ShapeDtypeStruct((n, m), jnp.float32),
    )(x, w)


def _pad_rows(x):
    return jnp.concatenate([x, jnp.zeros((NPAD - x.shape[0], x.shape[1]), x.dtype)], 0)


def _pad_edges(ei):
    pad = jnp.full((2, NS * EPT - ei.shape[1]), NPAD - 1, jnp.int32)
    e = jnp.concatenate([ei, pad], axis=1)
    return e[0].reshape(NS, CHUNKS, ECH), e[1].reshape(NS, CHUNKS, ECH)


def _pad_vec(v):
    return jnp.concatenate([v, jnp.full((NPAD - v.shape[0],), -1e9, v.dtype)])


def _edge_phase(ls, ld, src3, dst3, x2):
    m_shift = jnp.max(ls) + jnp.max(ld)
    m_splat = jnp.full((16,), m_shift, jnp.float32)
    agg2, den2 = _edge_sc(x2, _pad_vec(ls), _pad_vec(ld), m_splat,
                          src3, dst3)
    agg = jnp.concatenate([agg2[0], agg2[1]], axis=1).astype(jnp.float32)
    den = den2[0].reshape(NPAD)
    return agg / (den[:, None] + 1e-30)


def kernel(x_activity, x_resource_static, x_resource_dynamic, x_attribute,
           ei_follows, ei_has_rs, ei_rdelta, ei_has_rd, ei_has_attr,
           Wsrc, Wdst, Asrc, Adst, Bias, Wln, bln, Wfc, bfc):
    eis = [_pad_edges(ei) for ei in
           (ei_follows, ei_has_rs, ei_rdelta, ei_has_rd, ei_has_attr)]
    xs = [x_activity, x_resource_static, x_resource_dynamic, x_attribute]
    wsv = jnp.einsum('lrdh,lrh->lrd', Wsrc, Asrc)
    wdv = jnp.einsum('lrdh,lrh->lrd', Wdst, Adst)

    for l in range(L):
        cols = {t: [] for t in range(4)}
        for r in range(R):
            st, dt = REL[r]
            cols[st].append(('s', r, wsv[l, r]))
            cols[dt].append(('d', r, wdv[l, r]))
        ls = [None] * R
        ld = [None] * R
        for t in range(4):
            if not cols[t]:
                continue
            V = jnp.stack([c[2] for c in cols[t]], axis=1)
            Vp = jnp.concatenate([V, jnp.zeros((D, 128 - V.shape[1]), V.dtype)], 1)
            al = _mm(_pad_rows(xs[t]), Vp)[:N]
            for j, (kind, r, _) in enumerate(cols[t]):
                if kind == 's':
                    ls[r] = al[:, j]
                else:
                    ld[r] = al[:, j]

        xpads = {}
        for r in range(R):
            st = REL[r][0]
            if st not in xpads:
                xp = _pad_rows(xs[st]).astype(jnp.bfloat16)
                xpads[st] = jnp.stack([xp[:, :HD], xp[:, HD:]])
        outs = []
        for r in range(R):
            st, dt = REL[r]
            pooled = _edge_phase(ls[r], ld[r], eis[r][0], eis[r][1], xpads[st])
            outs.append(_mm(pooled, Wsrc[l, r])[:N] + Bias[l, r])
        xs = [
            jax.nn.relu(outs[0]),
            jax.nn.relu(outs[1]),
            jax.nn.relu((outs[2] + outs[3]) * 0.5),
            jax.nn.relu(outs[4]),
        ]

    feats = []
    for t in range(4):
        h = _mm(_pad_rows(xs[t]), Wln)[:N] + bln
        feats.append(jnp.mean(jax.nn.relu(h), axis=0))
    cat = jnp.concatenate(feats)
    logits = cat @ Wfc + bfc
    return jax.nn.softmax(logits, axis=-1)


# speedup vs baseline: 35.5068x; 1.0010x over previous
import functools

import jax
import jax.numpy as jnp
from jax import lax
from jax.experimental import pallas as pl
from jax.experimental.pallas import tpu as pltpu
from jax.experimental.pallas import tpu_sc as plsc

N = 10000
D = 128
HID = 128
OUT = 16
L = 2
R = 5
NPAD = 10240

NC = 2
NS = 16
HD = D // 2
ECH = 128
SCH = 4
SEDG = SCH * ECH
CHUNKS = 40
EPT = CHUNKS * SEDG
NROWS = NPAD // 16
RSTRIDE = NPAD // NS
DSTRIDE = NROWS // NS

_sc_mesh = plsc.VectorSubcoreMesh(
    core_axis_name="c", subcore_axis_name="s", num_cores=NC, num_subcores=NS)


@functools.partial(
    pl.kernel,
    out_type=(jax.ShapeDtypeStruct((NC, NPAD, HD), jnp.bfloat16),
              jax.ShapeDtypeStruct((NC, NROWS, 16), jnp.float32)),
    mesh=_sc_mesh,
    compiler_params=pltpu.CompilerParams(
        needs_layout_passes=False, use_tc_tiling_on_sc=False),
    scratch_types=(
        pltpu.VMEM((NPAD,), jnp.float32),
        pltpu.VMEM((NPAD,), jnp.float32),
        pltpu.VMEM((CHUNKS, SCH, ECH), jnp.int32),
        pltpu.VMEM((CHUNKS, SCH, ECH), jnp.int32),
        pltpu.VMEM((SEDG,), jnp.float32),
        pltpu.VMEM((2, SEDG, HD), jnp.bfloat16),
        pltpu.VMEM((NROWS, 16), jnp.float32),
        pltpu.VMEM((16,), jnp.float32),
        pltpu.VMEM((5, ECH), jnp.int32),
        pltpu.VMEM_SHARED((NPAD, HD), jnp.bfloat16),
        pltpu.VMEM_SHARED((NROWS, 16), jnp.float32),
        pltpu.SemaphoreType.DMA,
        pltpu.SemaphoreType.DMA,
        pltpu.SemaphoreType.DMA,
        pltpu.SemaphoreType.DMA,
    ),
)
def _edge_sc(x_hbm, ls_hbm, ld_hbm, m_hbm, src_hbm, dst_hbm,
             agg_out, den_out,
             ls_v, ld_v, src_v, dst_v, ex_v, rows_v, den_v, m_v, iota_v,
             agg_s, den_s, sem_g0, sem_g1, sem_s0, sem_s1):
    c = lax.axis_index("c")
    s = lax.axis_index("s")

    pltpu.sync_copy(ls_hbm, ls_v)
    pltpu.sync_copy(ld_hbm, ld_v)
    pltpu.sync_copy(m_hbm, m_v)
    pltpu.sync_copy(src_hbm.at[s], src_v)
    pltpu.sync_copy(dst_hbm.at[s], dst_v)

    zero16f = jnp.zeros((16,), jnp.float32)
    iota16 = lax.iota(jnp.int32, 16)

    zero32bf = jnp.zeros((32,), jnp.bfloat16)

    def _zrow(i, _):
        for f in range(HD // 32):
            rows_v[0, i, pl.ds(f * 32, 32)] = zero32bf
        return ()
    lax.fori_loop(0, SEDG, _zrow, ())

    def _zden(i, _):
        den_v[i, :] = zero16f
        return ()
    lax.fori_loop(0, NROWS, _zden, ())

    def _ziota(k, _):
        iota_v[k // 8, pl.ds((k % 8) * 16, 16)] = iota16 + k * 16
        return ()
    lax.fori_loop(0, 40, _ziota, ())

    for j in range(RSTRIDE // SEDG):
        pltpu.sync_copy(rows_v.at[0],
                        agg_s.at[pl.ds(s * RSTRIDE + j * SEDG, SEDG)])
    pltpu.sync_copy(rows_v.at[0, pl.ds(0, RSTRIDE % SEDG)],
                    agg_s.at[pl.ds(s * RSTRIDE + RSTRIDE // SEDG * SEDG,
                                   RSTRIDE % SEDG)])
    pltpu.sync_copy(den_v.at[pl.ds(s * DSTRIDE, DSTRIDE)],
                    den_s.at[pl.ds(s * DSTRIDE, DSTRIDE)])
    plsc.subcore_barrier()

    m = m_v[...]

    for k in range(SCH):
        pltpu.async_copy(x_hbm.at[c].at[src_v.at[0, k]],
                         rows_v.at[0, pl.ds(k * ECH, ECH)], sem_g0)

    @pl.loop(0, CHUNKS, step=2)
    def _pipe(ci):
        for b in range(2):
            cc = ci + b
            sg_cur = sem_g0 if b == 0 else sem_g1
            sg_oth = sem_g1 if b == 0 else sem_g0
            ss_cur = sem_s0 if b == 0 else sem_s1
            ss_oth = sem_s1 if b == 0 else sem_s0

            @pl.when(cc >= 1)
            def _():
                for k in range(SCH):
                    pltpu.make_async_copy(
                        rows_v.at[1 - b, pl.ds(k * ECH, ECH)],
                        agg_s.at[dst_v.at[cc - 1, k]], ss_oth).wait()

            @pl.when(cc + 1 < CHUNKS)
            def _():
                for k in range(SCH):
                    pltpu.async_copy(x_hbm.at[c].at[src_v.at[cc + 1, k]],
                                     rows_v.at[1 - b, pl.ds(k * ECH, ECH)],
                                     sg_oth)

            for k in range(SCH):
                for v in range(8):
                    s16 = src_v[cc, k, pl.ds(v * 16, 16)]
                    d16 = dst_v[cc, k, pl.ds(v * 16, 16)]
                    als = plsc.load_gather(ls_v, [s16])
                    ald = plsc.load_gather(ld_v, [d16])
                    e = als + ald
                    e = jnp.maximum(e, e * 0.2)
                    ex = jnp.exp(e - m)
                    ex_v[pl.ds(k * ECH + v * 16, 16)] = ex
                    plsc.addupdate_scatter(
                        den_v, [lax.shift_right_logical(d16, 4),
                                lax.bitwise_and(d16, 15)], ex)

            for k in range(SCH):
                pltpu.make_async_copy(x_hbm.at[c].at[src_v.at[cc, k]],
                                      rows_v.at[b, pl.ds(k * ECH, ECH)],
                                      sg_cur).wait()

            def row_body(i, _):
                spl = plsc.load_gather(ex_v, [jnp.zeros((16,), jnp.int32) + i])
                for f in range(HD // 32):
                    v32 = rows_v[b, i, pl.ds(f * 32, 32)]
                    lo, hi = plsc.unpack(v32, format=plsc.PackFormat.INTERLEAVED)
                    rows_v[b, i, pl.ds(f * 32, 32)] = plsc.pack(
                        lo * spl, hi * spl,
                        format=plsc.PackFormat.INTERLEAVED)
                return ()
            lax.fori_loop(0, SEDG, row_body, (), unroll=4)

            for k in range(SCH):
                pltpu.async_copy(rows_v.at[b, pl.ds(k * ECH, ECH)],
                                 agg_s.at[dst_v.at[cc, k]], ss_cur, add=True)

    for k in range(SCH):
        pltpu.make_async_copy(rows_v.at[1, pl.ds(k * ECH, ECH)],
                              agg_s.at[dst_v.at[0, k]], sem_s1).wait()

    plsc.subcore_barrier()
    for j in range(5):
        pltpu.sync_copy(den_v.at[pl.ds(j * ECH, ECH)],
                        den_s.at[iota_v.at[j]], add=True)
    plsc.subcore_barrier()

    pltpu.sync_copy(agg_s.at[pl.ds(s * RSTRIDE, RSTRIDE)],
                    agg_out.at[c, pl.ds(s * RSTRIDE, RSTRIDE)])
    pltpu.sync_copy(den_s.at[pl.ds(s * DSTRIDE, DSTRIDE)],
                    den_out.at[c, pl.ds(s * DSTRIDE, DSTRIDE)])

REL = ((0, 0), (0, 1), (2, 2), (0, 2), (0, 3))


def _mm_body(x_ref, w_ref, o_ref):
    o_ref[...] = jnp.dot(x_ref[...], w_ref[...], preferred_element_type=jnp.float32)


def _mm(x, w, bn=1024):
    n, k = x.shape
    m = w.shape[1]
    return pl.pallas_call(
        _mm_body,
        grid=(n // bn,),
        in_specs=[
            pl.BlockSpec((bn, k), lambda i: (i, 0)),
            pl.BlockSpec((k, m), lambda i: (0, 0)),
        ],
        out_specs=pl.BlockSpec((bn, m), lambda i: (i, 0)),
        out_shape=jax.ShapeDtypeStruct((n, m), jnp.float32),
    )(x, w)


def _pad_rows(x):
    return jnp.concatenate([x, jnp.zeros((NPAD - x.shape[0], x.shape[1]), x.dtype)], 0)


def _pad_edges(ei):
    pad = jnp.full((2, NS * EPT - ei.shape[1]), NPAD - 1, jnp.int32)
    e = jnp.concatenate([ei, pad], axis=1)
    return (e[0].reshape(NS, CHUNKS, SCH, ECH),
            e[1].reshape(NS, CHUNKS, SCH, ECH))


def _pad_vec(v):
    return jnp.concatenate([v, jnp.full((NPAD - v.shape[0],), -1e9, v.dtype)])


def _edge_phase(ls, ld, src3, dst3, x2):
    m_shift = jnp.max(ls) + jnp.max(ld)
    m_splat = jnp.full((16,), m_shift, jnp.float32)
    agg2, den2 = _edge_sc(x2, _pad_vec(ls), _pad_vec(ld), m_splat,
                          src3, dst3)
    agg = jnp.concatenate([agg2[0], agg2[1]], axis=1).astype(jnp.float32)
    den = den2[0].reshape(NPAD)
    return agg / (den[:, None] + 1e-30)


def kernel(x_activity, x_resource_static, x_resource_dynamic, x_attribute,
           ei_follows, ei_has_rs, ei_rdelta, ei_has_rd, ei_has_attr,
           Wsrc, Wdst, Asrc, Adst, Bias, Wln, bln, Wfc, bfc):
    eis = [_pad_edges(ei) for ei in
           (ei_follows, ei_has_rs, ei_rdelta, ei_has_rd, ei_has_attr)]
    xs = [x_activity, x_resource_static, x_resource_dynamic, x_attribute]
    wsv = jnp.einsum('lrdh,lrh->lrd', Wsrc, Asrc)
    wdv = jnp.einsum('lrdh,lrh->lrd', Wdst, Adst)

    for l in range(L):
        cols = {t: [] for t in range(4)}
        for r in range(R):
            st, dt = REL[r]
            cols[st].append(('s', r, wsv[l, r]))
            cols[dt].append(('d', r, wdv[l, r]))
        ls = [None] * R
        ld = [None] * R
        for t in range(4):
            if not cols[t]:
                continue
            V = jnp.stack([c[2] for c in cols[t]], axis=1)
            Vp = jnp.concatenate([V, jnp.zeros((D, 128 - V.shape[1]), V.dtype)], 1)
            al = _mm(_pad_rows(xs[t]), Vp)[:N]
            for j, (kind, r, _) in enumerate(cols[t]):
                if kind == 's':
                    ls[r] = al[:, j]
                else:
                    ld[r] = al[:, j]

        xpads = {}
        for r in range(R):
            st = REL[r][0]
            if st not in xpads:
                xp = _pad_rows(xs[st]).astype(jnp.bfloat16)
                xpads[st] = jnp.stack([xp[:, :HD], xp[:, HD:]])
        outs = []
        for r in range(R):
            st, dt = REL[r]
            pooled = _edge_phase(ls[r], ld[r], eis[r][0], eis[r][1], xpads[st])
            outs.append(_mm(pooled, Wsrc[l, r])[:N] + Bias[l, r])
        xs = [
            jax.nn.relu(outs[0]),
            jax.nn.relu(outs[1]),
            jax.nn.relu((outs[2] + outs[3]) * 0.5),
            jax.nn.relu(outs[4]),
        ]

    feats = []
    for t in range(4):
        h = _mm(_pad_rows(xs[t]), Wln)[:N] + bln
        feats.append(jnp.mean(jax.nn.relu(h), axis=0))
    cat = jnp.concatenate(feats)
    logits = cat @ Wfc + bfc
    return jax.nn.softmax(logits, axis=-1)


# speedup vs baseline: 45.8719x; 1.2919x over previous
import functools

import jax
import jax.numpy as jnp
from jax import lax
from jax.experimental import pallas as pl
from jax.experimental.pallas import tpu as pltpu
from jax.experimental.pallas import tpu_sc as plsc

N = 10000
D = 128
HID = 128
OUT = 16
L = 2
R = 5
NPAD = 10240

NC = 2
NS = 16
HD = D // 2
ECH = 128
SCH = 4
SEDG = SCH * ECH
CHUNKS = 40
EPT = CHUNKS * SEDG
NROWS = NPAD // 16
RSTRIDE = NPAD // NS
DSTRIDE = NROWS // NS

_sc_mesh = plsc.VectorSubcoreMesh(
    core_axis_name="c", subcore_axis_name="s", num_cores=NC, num_subcores=NS)


@functools.partial(
    pl.kernel,
    out_type=(jax.ShapeDtypeStruct((NC, NPAD, HD), jnp.bfloat16),
              jax.ShapeDtypeStruct((NC, NROWS, 16), jnp.float32)),
    mesh=_sc_mesh,
    compiler_params=pltpu.CompilerParams(
        needs_layout_passes=False, use_tc_tiling_on_sc=False),
    scratch_types=(
        pltpu.VMEM((NPAD,), jnp.float32),
        pltpu.VMEM((NPAD,), jnp.float32),
        pltpu.VMEM((CHUNKS, SCH, ECH), jnp.int32),
        pltpu.VMEM((CHUNKS, SCH, ECH), jnp.int32),
        pltpu.VMEM((SEDG,), jnp.float32),
        pltpu.VMEM((2, SEDG, HD), jnp.float8_e4m3fn),
        pltpu.VMEM((SEDG, HD), jnp.bfloat16),
        pltpu.VMEM((NROWS, 16), jnp.float32),
        pltpu.VMEM((16,), jnp.float32),
        pltpu.VMEM((5, ECH), jnp.int32),
        pltpu.VMEM_SHARED((NPAD, HD), jnp.bfloat16),
        pltpu.VMEM_SHARED((NROWS, 16), jnp.float32),
        pltpu.SemaphoreType.DMA,
        pltpu.SemaphoreType.DMA,
        pltpu.SemaphoreType.DMA,
        pltpu.SemaphoreType.DMA,
    ),
)
def _edge_sc(x_hbm, ls_hbm, ld_hbm, m_hbm, src_hbm, dst_hbm,
             agg_out, den_out,
             ls_v, ld_v, src_v, dst_v, ex_v, rows_v, sbuf, den_v, m_v, iota_v,
             agg_s, den_s, sem_g0, sem_g1, sem_s0, sem_s1):
    c = lax.axis_index("c")
    s = lax.axis_index("s")

    pltpu.sync_copy(ls_hbm, ls_v)
    pltpu.sync_copy(ld_hbm, ld_v)
    pltpu.sync_copy(m_hbm, m_v)
    pltpu.sync_copy(src_hbm.at[s], src_v)
    pltpu.sync_copy(dst_hbm.at[s], dst_v)

    zero16f = jnp.zeros((16,), jnp.float32)
    iota16 = lax.iota(jnp.int32, 16)

    zero32bf = jnp.zeros((32,), jnp.bfloat16)

    def _zrow(i, _):
        for f in range(HD // 32):
            sbuf[i, pl.ds(f * 32, 32)] = zero32bf
        return ()
    lax.fori_loop(0, SEDG, _zrow, ())

    def _zden(i, _):
        den_v[i, :] = zero16f
        return ()
    lax.fori_loop(0, NROWS, _zden, ())

    def _ziota(k, _):
        iota_v[k // 8, pl.ds((k % 8) * 16, 16)] = iota16 + k * 16
        return ()
    lax.fori_loop(0, 40, _ziota, ())

    for j in range(RSTRIDE // SEDG):
        pltpu.sync_copy(sbuf, agg_s.at[pl.ds(s * RSTRIDE + j * SEDG, SEDG)])
    pltpu.sync_copy(sbuf.at[pl.ds(0, RSTRIDE % SEDG)],
                    agg_s.at[pl.ds(s * RSTRIDE + RSTRIDE // SEDG * SEDG,
                                   RSTRIDE % SEDG)])
    pltpu.sync_copy(den_v.at[pl.ds(s * DSTRIDE, DSTRIDE)],
                    den_s.at[pl.ds(s * DSTRIDE, DSTRIDE)])
    plsc.subcore_barrier()

    m = m_v[...]

    for k in range(SCH):
        pltpu.async_copy(x_hbm.at[c].at[src_v.at[0, k]],
                         rows_v.at[0, pl.ds(k * ECH, ECH)], sem_g0)

    @pl.loop(0, CHUNKS, step=2)
    def _pipe(ci):
        for b in range(2):
            cc = ci + b
            sg_cur = sem_g0 if b == 0 else sem_g1
            sg_oth = sem_g1 if b == 0 else sem_g0

            @pl.when(cc + 1 < CHUNKS)
            def _():
                for k in range(SCH):
                    pltpu.async_copy(x_hbm.at[c].at[src_v.at[cc + 1, k]],
                                     rows_v.at[1 - b, pl.ds(k * ECH, ECH)],
                                     sg_oth)

            for k in range(SCH):
                for v in range(8):
                    s16 = src_v[cc, k, pl.ds(v * 16, 16)]
                    d16 = dst_v[cc, k, pl.ds(v * 16, 16)]
                    als = plsc.load_gather(ls_v, [s16])
                    ald = plsc.load_gather(ld_v, [d16])
                    e = als + ald
                    e = jnp.maximum(e, e * 0.2)
                    ex = jnp.exp(e - m)
                    ex_v[pl.ds(k * ECH + v * 16, 16)] = ex
                    plsc.addupdate_scatter(
                        den_v, [lax.shift_right_logical(d16, 4),
                                lax.bitwise_and(d16, 15)], ex)

            for k in range(SCH):
                pltpu.make_async_copy(x_hbm.at[c].at[src_v.at[cc, k]],
                                      rows_v.at[b, pl.ds(k * ECH, ECH)],
                                      sg_cur).wait()

            @pl.when(cc >= 1)
            def _():
                for k in range(SCH):
                    pltpu.make_async_copy(sbuf.at[pl.ds(k * ECH, ECH)],
                                          agg_s.at[dst_v.at[cc - 1, k]],
                                          sem_s0).wait()

            def row_body(i, _):
                spl = plsc.load_gather(ex_v, [jnp.zeros((16,), jnp.int32) + i])
                splbf = plsc.pack(spl, spl,
                                  format=plsc.PackFormat.INTERLEAVED)
                v64 = rows_v[b, i, pl.ds(0, 64)]
                lo, hi = plsc.unpack(v64,
                                     format=plsc.PackFormat.INTERLEAVED,
                                     preferred_element_type=jnp.bfloat16)
                sbuf[i, pl.ds(0, 32)] = lo * splbf
                sbuf[i, pl.ds(32, 32)] = hi * splbf
                return ()
            lax.fori_loop(0, SEDG, row_body, (), unroll=4)

            for k in range(SCH):
                pltpu.async_copy(sbuf.at[pl.ds(k * ECH, ECH)],
                                 agg_s.at[dst_v.at[cc, k]], sem_s0, add=True)

    for k in range(SCH):
        pltpu.make_async_copy(sbuf.at[pl.ds(k * ECH, ECH)],
                              agg_s.at[dst_v.at[0, k]], sem_s0).wait()

    plsc.subcore_barrier()
    for j in range(5):
        pltpu.sync_copy(den_v.at[pl.ds(j * ECH, ECH)],
                        den_s.at[iota_v.at[j]], add=True)
    plsc.subcore_barrier()

    pltpu.sync_copy(agg_s.at[pl.ds(s * RSTRIDE, RSTRIDE)],
                    agg_out.at[c, pl.ds(s * RSTRIDE, RSTRIDE)])
    pltpu.sync_copy(den_s.at[pl.ds(s * DSTRIDE, DSTRIDE)],
                    den_out.at[c, pl.ds(s * DSTRIDE, DSTRIDE)])

REL = ((0, 0), (0, 1), (2, 2), (0, 2), (0, 3))


def _mm_body(x_ref, w_ref, o_ref):
    o_ref[...] = jnp.dot(x_ref[...], w_ref[...], preferred_element_type=jnp.float32)


def _mm(x, w, bn=1024):
    n, k = x.shape
    m = w.shape[1]
    return pl.pallas_call(
        _mm_body,
        grid=(n // bn,),
        in_specs=[
            pl.BlockSpec((bn, k), lambda i: (i, 0)),
            pl.BlockSpec((k, m), lambda i: (0, 0)),
        ],
        out_specs=pl.BlockSpec((bn, m), lambda i: (i, 0)),
        out_shape=jax.ShapeDtypeStruct((n, m), jnp.float32),
    )(x, w)


def _pad_rows(x):
    return jnp.concatenate([x, jnp.zeros((NPAD - x.shape[0], x.shape[1]), x.dtype)], 0)


def _pad_edges(ei):
    pad = jnp.full((2, NS * EPT - ei.shape[1]), NPAD - 1, jnp.int32)
    e = jnp.concatenate([ei, pad], axis=1)
    return (e[0].reshape(NS, CHUNKS, SCH, ECH),
            e[1].reshape(NS, CHUNKS, SCH, ECH))


def _pad_vec(v):
    return jnp.concatenate([v, jnp.full((NPAD - v.shape[0],), -1e9, v.dtype)])


def _edge_phase(ls, ld, src3, dst3, x2):
    m_shift = jnp.max(ls) + jnp.max(ld)
    m_splat = jnp.full((16,), m_shift, jnp.float32)
    agg2, den2 = _edge_sc(x2, _pad_vec(ls), _pad_vec(ld), m_splat,
                          src3, dst3)
    agg = jnp.concatenate([agg2[0], agg2[1]], axis=1).astype(jnp.float32)
    den = den2[0].reshape(NPAD)
    return agg / (den[:, None] + 1e-30)


def kernel(x_activity, x_resource_static, x_resource_dynamic, x_attribute,
           ei_follows, ei_has_rs, ei_rdelta, ei_has_rd, ei_has_attr,
           Wsrc, Wdst, Asrc, Adst, Bias, Wln, bln, Wfc, bfc):
    eis = [_pad_edges(ei) for ei in
           (ei_follows, ei_has_rs, ei_rdelta, ei_has_rd, ei_has_attr)]
    xs = [x_activity, x_resource_static, x_resource_dynamic, x_attribute]
    wsv = jnp.einsum('lrdh,lrh->lrd', Wsrc, Asrc)
    wdv = jnp.einsum('lrdh,lrh->lrd', Wdst, Adst)

    for l in range(L):
        cols = {t: [] for t in range(4)}
        for r in range(R):
            st, dt = REL[r]
            cols[st].append(('s', r, wsv[l, r]))
            cols[dt].append(('d', r, wdv[l, r]))
        ls = [None] * R
        ld = [None] * R
        for t in range(4):
            if not cols[t]:
                continue
            V = jnp.stack([c[2] for c in cols[t]], axis=1)
            Vp = jnp.concatenate([V, jnp.zeros((D, 128 - V.shape[1]), V.dtype)], 1)
            al = _mm(_pad_rows(xs[t]), Vp)[:N]
            for j, (kind, r, _) in enumerate(cols[t]):
                if kind == 's':
                    ls[r] = al[:, j]
                else:
                    ld[r] = al[:, j]

        xpads = {}
        for r in range(R):
            st = REL[r][0]
            if st not in xpads:
                xp = _pad_rows(xs[st])
                halves = []
                for h in (xp[:, :HD], xp[:, HD:]):
                    halves.append(jnp.stack([h[:, :HD // 2], h[:, HD // 2:]],
                                            axis=-1).reshape(NPAD, HD))
                xpads[st] = jnp.stack(halves).astype(jnp.float8_e4m3fn)
        outs = []
        for r in range(R):
            st, dt = REL[r]
            pooled = _edge_phase(ls[r], ld[r], eis[r][0], eis[r][1], xpads[st])
            outs.append(_mm(pooled, Wsrc[l, r])[:N] + Bias[l, r])
        xs = [
            jax.nn.relu(outs[0]),
            jax.nn.relu(outs[1]),
            jax.nn.relu((outs[2] + outs[3]) * 0.5),
            jax.nn.relu(outs[4]),
        ]

    feats = []
    for t in range(4):
        h = _mm(_pad_rows(xs[t]), Wln)[:N] + bln
        feats.append(jnp.mean(jax.nn.relu(h), axis=0))
    cat = jnp.concatenate(feats)
    logits = cat @ Wfc + bfc
    return jax.nn.softmax(logits, axis=-1)


# speedup vs baseline: 65.8774x; 1.4361x over previous
import functools

import jax
import jax.numpy as jnp
from jax import lax
from jax.experimental import pallas as pl
from jax.experimental.pallas import tpu as pltpu
from jax.experimental.pallas import tpu_sc as plsc

N = 10000
D = 128
HID = 128
OUT = 16
L = 2
R = 5
NPAD = 10240

NC = 2
NS = 16
HD = D // 2
ECH = 128
SCH = 4
SEDG = SCH * ECH
CHUNKS = 40
EPT = CHUNKS * SEDG
NROWS = NPAD // 16
RSTRIDE = NPAD // NS
DSTRIDE = NROWS // NS

_sc_mesh = plsc.VectorSubcoreMesh(
    core_axis_name="c", subcore_axis_name="s", num_cores=NC, num_subcores=NS)


@functools.partial(
    pl.kernel,
    out_type=(jax.ShapeDtypeStruct((NC, NPAD, HD), jnp.bfloat16),
              jax.ShapeDtypeStruct((NC, NROWS, 16), jnp.float32)),
    mesh=_sc_mesh,
    compiler_params=pltpu.CompilerParams(
        needs_layout_passes=False, use_tc_tiling_on_sc=False),
    scratch_types=(
        pltpu.VMEM((NPAD,), jnp.float32),
        pltpu.VMEM((NPAD,), jnp.float32),
        pltpu.VMEM((CHUNKS, SCH, ECH), jnp.int32),
        pltpu.VMEM((CHUNKS, SCH, ECH), jnp.int32),
        pltpu.VMEM((SEDG,), jnp.float32),
        pltpu.VMEM((2, SEDG, HD), jnp.float8_e4m3fn),
        pltpu.VMEM((SEDG, HD), jnp.bfloat16),
        pltpu.VMEM((NROWS, 16), jnp.float32),
        pltpu.VMEM((16,), jnp.float32),
        pltpu.VMEM((5, ECH), jnp.int32),
        pltpu.VMEM_SHARED((NPAD, HD), jnp.bfloat16),
        pltpu.VMEM_SHARED((NROWS, 16), jnp.float32),
        pltpu.SemaphoreType.DMA,
        pltpu.SemaphoreType.DMA,
        pltpu.SemaphoreType.DMA,
        pltpu.SemaphoreType.DMA,
    ),
)
def _edge_sc(x_hbm, ls_hbm, ld_hbm, m_hbm, src_hbm, dst_hbm,
             agg_out, den_out,
             ls_v, ld_v, src_v, dst_v, ex_v, rows_v, sbuf, den_v, m_v, iota_v,
             agg_s, den_s, sem_g0, sem_g1, sem_s0, sem_s1):
    c = lax.axis_index("c")
    s = lax.axis_index("s")

    pltpu.sync_copy(ls_hbm, ls_v)
    pltpu.sync_copy(ld_hbm, ld_v)
    pltpu.sync_copy(m_hbm, m_v)
    pltpu.sync_copy(src_hbm.at[s], src_v)
    pltpu.sync_copy(dst_hbm.at[s], dst_v)

    zero16f = jnp.zeros((16,), jnp.float32)
    iota16 = lax.iota(jnp.int32, 16)

    zero32bf = jnp.zeros((32,), jnp.bfloat16)

    def _zrow(i, _):
        for f in range(HD // 32):
            sbuf[i, pl.ds(f * 32, 32)] = zero32bf
        return ()
    lax.fori_loop(0, SEDG, _zrow, ())

    def _zden(i, _):
        den_v[i, :] = zero16f
        return ()
    lax.fori_loop(0, NROWS, _zden, ())

    def _ziota(k, _):
        iota_v[k // 8, pl.ds((k % 8) * 16, 16)] = iota16 + k * 16
        return ()
    lax.fori_loop(0, 40, _ziota, ())

    for j in range(RSTRIDE // SEDG):
        pltpu.sync_copy(sbuf, agg_s.at[pl.ds(s * RSTRIDE + j * SEDG, SEDG)])
    pltpu.sync_copy(sbuf.at[pl.ds(0, RSTRIDE % SEDG)],
                    agg_s.at[pl.ds(s * RSTRIDE + RSTRIDE // SEDG * SEDG,
                                   RSTRIDE % SEDG)])
    pltpu.sync_copy(den_v.at[pl.ds(s * DSTRIDE, DSTRIDE)],
                    den_s.at[pl.ds(s * DSTRIDE, DSTRIDE)])
    plsc.subcore_barrier()

    m = m_v[...]

    for k in range(SCH):
        pltpu.async_copy(x_hbm.at[c].at[src_v.at[0, k]],
                         rows_v.at[0, pl.ds(k * ECH, ECH)], sem_g0)

    @pl.loop(0, CHUNKS, step=2)
    def _pipe(ci):
        for b in range(2):
            cc = ci + b
            sg_cur = sem_g0 if b == 0 else sem_g1
            sg_oth = sem_g1 if b == 0 else sem_g0

            @pl.when(cc + 1 < CHUNKS)
            def _():
                for k in range(SCH):
                    pltpu.async_copy(x_hbm.at[c].at[src_v.at[cc + 1, k]],
                                     rows_v.at[1 - b, pl.ds(k * ECH, ECH)],
                                     sg_oth)

            for k in range(SCH):
                for v in range(8):
                    s16 = src_v[cc, k, pl.ds(v * 16, 16)]
                    d16 = dst_v[cc, k, pl.ds(v * 16, 16)]
                    als = plsc.load_gather(ls_v, [s16])
                    ald = plsc.load_gather(ld_v, [d16])
                    e = als + ald
                    e = jnp.maximum(e, e * 0.2)
                    ex = jnp.exp(e - m)
                    ex_v[pl.ds(k * ECH + v * 16, 16)] = ex
                    plsc.addupdate_scatter(
                        den_v, [lax.shift_right_logical(d16, 4),
                                lax.bitwise_and(d16, 15)], ex)

            for k in range(SCH):
                pltpu.make_async_copy(x_hbm.at[c].at[src_v.at[cc, k]],
                                      rows_v.at[b, pl.ds(k * ECH, ECH)],
                                      sg_cur).wait()

            @pl.when(cc >= 1)
            def _():
                for k in range(SCH):
                    pltpu.make_async_copy(sbuf.at[pl.ds(k * ECH, ECH)],
                                          agg_s.at[dst_v.at[cc - 1, k]],
                                          sem_s0).wait()

            @plsc.parallel_loop(0, SEDG, unroll=8)
            def _scale(i):
                spl = plsc.load_gather(ex_v, [jnp.zeros((16,), jnp.int32) + i])
                splbf = plsc.pack(spl, spl,
                                  format=plsc.PackFormat.INTERLEAVED)
                v64 = rows_v[b, i, pl.ds(0, 64)]
                lo, hi = plsc.unpack(v64,
                                     format=plsc.PackFormat.INTERLEAVED,
                                     preferred_element_type=jnp.bfloat16)
                sbuf[i, pl.ds(0, 32)] = lo * splbf
                sbuf[i, pl.ds(32, 32)] = hi * splbf

            for k in range(SCH):
                pltpu.async_copy(sbuf.at[pl.ds(k * ECH, ECH)],
                                 agg_s.at[dst_v.at[cc, k]], sem_s0, add=True)

    for k in range(SCH):
        pltpu.make_async_copy(sbuf.at[pl.ds(k * ECH, ECH)],
                              agg_s.at[dst_v.at[0, k]], sem_s0).wait()

    plsc.subcore_barrier()
    for j in range(5):
        pltpu.sync_copy(den_v.at[pl.ds(j * ECH, ECH)],
                        den_s.at[iota_v.at[j]], add=True)
    plsc.subcore_barrier()

    pltpu.sync_copy(agg_s.at[pl.ds(s * RSTRIDE, RSTRIDE)],
                    agg_out.at[c, pl.ds(s * RSTRIDE, RSTRIDE)])
    pltpu.sync_copy(den_s.at[pl.ds(s * DSTRIDE, DSTRIDE)],
                    den_out.at[c, pl.ds(s * DSTRIDE, DSTRIDE)])

REL = ((0, 0), (0, 1), (2, 2), (0, 2), (0, 3))


def _mm_body(x_ref, w_ref, o_ref):
    o_ref[...] = jnp.dot(x_ref[...], w_ref[...], preferred_element_type=jnp.float32)


def _mm(x, w, bn=1024):
    n, k = x.shape
    m = w.shape[1]
    return pl.pallas_call(
        _mm_body,
        grid=(n // bn,),
        in_specs=[
            pl.BlockSpec((bn, k), lambda i: (i, 0)),
            pl.BlockSpec((k, m), lambda i: (0, 0)),
        ],
        out_specs=pl.BlockSpec((bn, m), lambda i: (i, 0)),
        out_shape=jax.ShapeDtypeStruct((n, m), jnp.float32),
    )(x, w)


def _pad_rows(x):
    return jnp.concatenate([x, jnp.zeros((NPAD - x.shape[0], x.shape[1]), x.dtype)], 0)


def _pad_edges(ei):
    pad = jnp.full((2, NS * EPT - ei.shape[1]), NPAD - 1, jnp.int32)
    e = jnp.concatenate([ei, pad], axis=1)
    return (e[0].reshape(NS, CHUNKS, SCH, ECH),
            e[1].reshape(NS, CHUNKS, SCH, ECH))


def _pad_vec(v):
    return jnp.concatenate([v, jnp.full((NPAD - v.shape[0],), -1e9, v.dtype)])


def _edge_phase(ls, ld, src3, dst3, x2):
    m_shift = jnp.max(ls) + jnp.max(ld)
    m_splat = jnp.full((16,), m_shift, jnp.float32)
    agg2, den2 = _edge_sc(x2, _pad_vec(ls), _pad_vec(ld), m_splat,
                          src3, dst3)
    agg = jnp.concatenate([agg2[0], agg2[1]], axis=1).astype(jnp.float32)
    den = den2[0].reshape(NPAD)
    return agg / (den[:, None] + 1e-30)


def kernel(x_activity, x_resource_static, x_resource_dynamic, x_attribute,
           ei_follows, ei_has_rs, ei_rdelta, ei_has_rd, ei_has_attr,
           Wsrc, Wdst, Asrc, Adst, Bias, Wln, bln, Wfc, bfc):
    eis = [_pad_edges(ei) for ei in
           (ei_follows, ei_has_rs, ei_rdelta, ei_has_rd, ei_has_attr)]
    xs = [x_activity, x_resource_static, x_resource_dynamic, x_attribute]
    wsv = jnp.einsum('lrdh,lrh->lrd', Wsrc, Asrc)
    wdv = jnp.einsum('lrdh,lrh->lrd', Wdst, Adst)

    for l in range(L):
        cols = {t: [] for t in range(4)}
        for r in range(R):
            st, dt = REL[r]
            cols[st].append(('s', r, wsv[l, r]))
            cols[dt].append(('d', r, wdv[l, r]))
        ls = [None] * R
        ld = [None] * R
        for t in range(4):
            if not cols[t]:
                continue
            V = jnp.stack([c[2] for c in cols[t]], axis=1)
            Vp = jnp.concatenate([V, jnp.zeros((D, 128 - V.shape[1]), V.dtype)], 1)
            al = _mm(_pad_rows(xs[t]), Vp)[:N]
            for j, (kind, r, _) in enumerate(cols[t]):
                if kind == 's':
                    ls[r] = al[:, j]
                else:
                    ld[r] = al[:, j]

        xpads = {}
        for r in range(R):
            st = REL[r][0]
            if st not in xpads:
                xp = _pad_rows(xs[st])
                halves = []
                for h in (xp[:, :HD], xp[:, HD:]):
                    halves.append(jnp.stack([h[:, :HD // 2], h[:, HD // 2:]],
                                            axis=-1).reshape(NPAD, HD))
                xpads[st] = jnp.stack(halves).astype(jnp.float8_e4m3fn)
        outs = []
        for r in range(R):
            st, dt = REL[r]
            pooled = _edge_phase(ls[r], ld[r], eis[r][0], eis[r][1], xpads[st])
            outs.append(_mm(pooled, Wsrc[l, r])[:N] + Bias[l, r])
        xs = [
            jax.nn.relu(outs[0]),
            jax.nn.relu(outs[1]),
            jax.nn.relu((outs[2] + outs[3]) * 0.5),
            jax.nn.relu(outs[4]),
        ]

    feats = []
    for t in range(4):
        h = _mm(_pad_rows(xs[t]), Wln)[:N] + bln
        feats.append(jnp.mean(jax.nn.relu(h), axis=0))
    cat = jnp.concatenate(feats)
    logits = cat @ Wfc + bfc
    return jax.nn.softmax(logits, axis=-1)
